# Initial kernel scaffold; baseline (speedup 1.0000x reference)
import jax
import jax.numpy as jnp
from jax.experimental import pallas as pl


def kernel(input1, input2):
    raise NotImplementedError("write your pallas kernel here")


# speedup vs baseline: 4.5488x; 4.5488x over previous
import functools

import jax
import jax.numpy as jnp
from jax import lax
from jax.experimental import pallas as pl
from jax.experimental.pallas import tpu as pltpu
from jax.experimental.pallas import tpu_sc as plsc

EPS = 1e-09
J = 16384
K = 16
A = 128
NC, NS = 2, 16
NW = NC * NS
ROWS_W = J // NW
RB = 8
GN = RB * K
NBATCH = ROWS_W // RB
NCH = A // 16


def _rsqrt(x):
    i = lax.bitcast_convert_type(x, jnp.int32)
    i = 0x5F3759DF - lax.shift_right_logical(i, 1)
    y = lax.bitcast_convert_type(i, jnp.float32)
    for _ in range(3):
        y = y * (1.5 - 0.5 * x * y * y)
    return y


_MESH = plsc.VectorSubcoreMesh(
    core_axis_name="c", subcore_axis_name="s", num_cores=NC, num_subcores=NS
)


@functools.partial(
    pl.kernel,
    out_type=jax.ShapeDtypeStruct((J * K,), jnp.float32),
    mesh=_MESH,
    compiler_params=pltpu.CompilerParams(needs_layout_passes=False),
    scratch_types=[
        pltpu.VMEM((ROWS_W * K,), jnp.int32),
        pltpu.VMEM((ROWS_W, A), jnp.float32),
        pltpu.VMEM((GN, A), jnp.float32),
        pltpu.VMEM((GN, A), jnp.float32),
        pltpu.VMEM((K * 16,), jnp.float32),
        pltpu.VMEM((ROWS_W * K,), jnp.float32),
        pltpu.SemaphoreType.DMA,
        pltpu.SemaphoreType.DMA,
    ],
)
def _buildk_sc(uu, idxf, out, idx_v, q_v, nb0, nb1, st_v, w_v, sem0, sem1):
    wid = lax.axis_index("s") * NC + lax.axis_index("c")
    row0 = wid * ROWS_W

    pltpu.sync_copy(idxf.at[pl.ds(row0 * K, ROWS_W * K)], idx_v)
    pltpu.sync_copy(uu.at[pl.ds(row0, ROWS_W)], q_v)

    nbs = (nb0, nb1)
    sems = (sem0, sem1)

    def gather_start(g, b):
        off = pl.multiple_of(g * GN, GN)
        pltpu.async_copy(uu.at[idx_v.at[pl.ds(off, GN)]], nbs[b], sems[b])

    def gather_wait(g, b):
        off = pl.multiple_of(g * GN, GN)
        pltpu.make_async_copy(
            uu.at[idx_v.at[pl.ds(off, GN)]], nbs[b], sems[b]
        ).wait()

    lane = lax.iota(jnp.int32, 16)

    def compute_batch(g, nb):
        @pl.loop(0, RB)
        def _row(r):
            row = g * RB + r
            qc = [q_v[row, pl.ds(c * 16, 16)] for c in range(NCH)]
            for i in range(K):
                nrow = r * K + i
                diff = nb[nrow, pl.ds(0, 16)] - qc[0]
                acc = diff * diff
                for c in range(1, NCH):
                    diff = nb[nrow, pl.ds(c * 16, 16)] - qc[c]
                    acc = acc + diff * diff
                st_v[pl.ds(i * 16, 16)] = acc
            dsum = plsc.load_gather(st_v, [lane * 16])
            for l in range(1, 16):
                dsum = dsum + plsc.load_gather(st_v, [lane * 16 + l])
            msd = dsum * (1.0 / A) + EPS
            dvec = -(msd * _rsqrt(msd))
            e = jnp.exp(dvec - jnp.max(dvec))
            wv = e / jnp.sum(e)
            w_v[pl.ds(pl.multiple_of(row * K, K), K)] = wv

    gather_start(0, 0)
    gather_start(1, 1)

    @pl.loop(0, NBATCH, step=2)
    def _pair(g):
        for b in range(2):
            gg = g + b
            gather_wait(gg, b)
            compute_batch(gg, nbs[b])

            @pl.when(gg + 2 < NBATCH)
            def _():
                gather_start(gg + 2, b)

    pltpu.sync_copy(w_v, out.at[pl.ds(row0 * K, ROWS_W * K)])


def kernel(input1, input2):
    a = input1.shape[1]
    uu = input1.reshape(a, -1).T.astype(jnp.float32)
    idxf = input2.astype(jnp.int32).reshape(-1)
    w = _buildk_sc(uu, idxf)
    return w.reshape(J, K)


# speedup vs baseline: 6.6653x; 1.4653x over previous
import functools

import jax
import jax.numpy as jnp
from jax import lax
from jax.experimental import pallas as pl
from jax.experimental.pallas import tpu as pltpu
from jax.experimental.pallas import tpu_sc as plsc

EPS = 1e-09
J = 16384
K = 16
A = 128
NC, NS = 2, 16
NW = NC * NS
ROWS_W = J // NW
RB = 8
GN = RB * K
NBATCH = ROWS_W // RB
NCH = A // 16


def _rsqrt(x):
    i = lax.bitcast_convert_type(x, jnp.int32)
    i = 0x5F3759DF - lax.shift_right_logical(i, 1)
    y = lax.bitcast_convert_type(i, jnp.float32)
    for _ in range(3):
        y = y * (1.5 - 0.5 * x * y * y)
    return y


_MESH = plsc.VectorSubcoreMesh(
    core_axis_name="c", subcore_axis_name="s", num_cores=NC, num_subcores=NS
)


@functools.partial(
    pl.kernel,
    out_type=jax.ShapeDtypeStruct((J * K,), jnp.float32),
    mesh=_MESH,
    compiler_params=pltpu.CompilerParams(needs_layout_passes=False),
    scratch_types=[
        pltpu.VMEM((ROWS_W * K,), jnp.int32),
        pltpu.VMEM((ROWS_W, A), jnp.float32),
        pltpu.VMEM((GN, A), jnp.float32),
        pltpu.VMEM((GN, A), jnp.float32),
        pltpu.VMEM((RB * K * 16,), jnp.float32),
        pltpu.VMEM((ROWS_W * K,), jnp.float32),
        pltpu.SemaphoreType.DMA,
        pltpu.SemaphoreType.DMA,
    ],
)
def _buildk_sc(uu, idxf, out, idx_v, q_v, nb0, nb1, st_v, w_v, sem0, sem1):
    wid = lax.axis_index("s") * NC + lax.axis_index("c")
    row0 = wid * ROWS_W

    pltpu.sync_copy(idxf.at[pl.ds(row0 * K, ROWS_W * K)], idx_v)
    pltpu.sync_copy(uu.at[pl.ds(row0, ROWS_W)], q_v)

    nbs = (nb0, nb1)
    sems = (sem0, sem1)

    def gather_start(g, b):
        off = pl.multiple_of(g * GN, GN)
        pltpu.async_copy(uu.at[idx_v.at[pl.ds(off, GN)]], nbs[b], sems[b])

    def gather_wait(g, b):
        off = pl.multiple_of(g * GN, GN)
        pltpu.make_async_copy(
            uu.at[idx_v.at[pl.ds(off, GN)]], nbs[b], sems[b]
        ).wait()

    lane = lax.iota(jnp.int32, 16)

    def _treesum(vals):
        vals = list(vals)
        while len(vals) > 1:
            vals = [
                vals[j] + vals[j + 1] if j + 1 < len(vals) else vals[j]
                for j in range(0, len(vals), 2)
            ]
        return vals[0]

    def compute_batch(g, nb):
        @plsc.parallel_loop(0, RB, unroll=2)
        def _row(r):
            row = g * RB + r
            sbase = r * (K * 16)
            qc = [q_v[row, pl.ds(c * 16, 16)] for c in range(NCH)]
            for i in range(K):
                nrow = r * K + i
                sq = []
                for c in range(NCH):
                    diff = nb[nrow, pl.ds(c * 16, 16)] - qc[c]
                    sq.append(diff * diff)
                st_v[pl.ds(pl.multiple_of(sbase + i * 16, 16), 16)] = _treesum(sq)
            dsum = _treesum(
                [
                    plsc.load_gather(st_v, [sbase + lane * 16 + l])
                    for l in range(16)
                ]
            )
            msd = dsum * (1.0 / A) + EPS
            dvec = -(msd * _rsqrt(msd))
            e = jnp.exp(dvec - jnp.max(dvec))
            wv = e / jnp.sum(e)
            w_v[pl.ds(pl.multiple_of(row * K, K), K)] = wv

    gather_start(0, 0)
    gather_start(1, 1)

    @pl.loop(0, NBATCH, step=2)
    def _pair(g):
        for b in range(2):
            gg = g + b
            gather_wait(gg, b)
            compute_batch(gg, nbs[b])

            @pl.when(gg + 2 < NBATCH)
            def _():
                gather_start(gg + 2, b)

    pltpu.sync_copy(w_v, out.at[pl.ds(row0 * K, ROWS_W * K)])


def kernel(input1, input2):
    a = input1.shape[1]
    uu = input1.reshape(a, -1).T.astype(jnp.float32)
    idxf = input2.astype(jnp.int32).reshape(-1)
    w = _buildk_sc(uu, idxf)
    return w.reshape(J, K)
